# baseline (speedup 1.0000x reference)
import jax
import jax.numpy as jnp
from jax import lax
from jax.experimental import pallas as pl
from jax.experimental.pallas import tpu as pltpu
from jax.experimental.pallas import tpu_sc as plsc

_NW = 32
_CH = 200
_NBUF = 4
_LAG = 2


def _sc_copy_body(u_src, i_src, u_dst, i_dst, buf,
                  si0, si1, si2, si3, so0, so1, so2, so3):
    sem_in = (si0, si1, si2, si3)
    sem_out = (so0, so1, so2, so3)
    wid = lax.axis_index("s") * 2 + lax.axis_index("c")

    chunks = []
    for src, dst, n in ((u_src, u_dst, 100000), (i_src, i_dst, 1000000)):
        n_chunks = n // _CH
        per_w = -(-n_chunks // _NW)
        for k in range(per_w):
            cid = jnp.minimum(wid + _NW * k, n_chunks - 1)
            off = pl.multiple_of(cid * _CH, 8)
            chunks.append((src, dst, off))
    T = len(chunks)

    def copy_in(c):
        s, _, off = chunks[c]
        b = c % _NBUF
        return pltpu.make_async_copy(
            s.at[pl.ds(off, _CH), :], buf.at[b], sem_in[b])

    def copy_out(c):
        _, d, off = chunks[c]
        b = c % _NBUF
        return pltpu.make_async_copy(
            buf.at[b], d.at[pl.ds(off, _CH), :], sem_out[b])

    out_waited = [False] * T
    for b in range(min(_NBUF, T)):
        copy_in(b).start()
    for c in range(T):
        r = c - _LAG
        if 0 <= r and r + _NBUF < T:
            copy_out(r).wait()
            out_waited[r] = True
            copy_in(r + _NBUF).start()
        copy_in(c).wait()
        copy_out(c).start()
    for c in range(T):
        if not out_waited[c]:
            copy_out(c).wait()


def kernel(W_user, W_item):
    mesh = plsc.VectorSubcoreMesh(core_axis_name="c", subcore_axis_name="s")
    f = pl.kernel(
        _sc_copy_body,
        out_type=(
            jax.ShapeDtypeStruct(W_user.shape, W_user.dtype),
            jax.ShapeDtypeStruct(W_item.shape, W_item.dtype),
        ),
        mesh=mesh,
        scratch_types=[
            pltpu.VMEM((_NBUF, _CH, 64), jnp.float32),
            pltpu.SemaphoreType.DMA,
            pltpu.SemaphoreType.DMA,
            pltpu.SemaphoreType.DMA,
            pltpu.SemaphoreType.DMA,
            pltpu.SemaphoreType.DMA,
            pltpu.SemaphoreType.DMA,
            pltpu.SemaphoreType.DMA,
            pltpu.SemaphoreType.DMA,
        ],
    )
    return f(W_user, W_item)


# speedup vs baseline: 1.0295x; 1.0295x over previous
import jax
import jax.numpy as jnp
from jax import lax
from jax.experimental import pallas as pl
from jax.experimental.pallas import tpu as pltpu
from jax.experimental.pallas import tpu_sc as plsc

_NW = 32
_CH = 400
_NBUF = 2


def _sc_copy_body(i_src, i_dst, buf, si0, si1, so0, so1):
    sem_in = (si0, si1)
    sem_out = (so0, so1)
    wid = lax.axis_index("s") * 2 + lax.axis_index("c")

    chunks = []
    n = 1000000
    n_chunks = n // _CH
    per_w = -(-n_chunks // _NW)
    for k in range(per_w):
        cid = jnp.minimum(wid + _NW * k, n_chunks - 1)
        off = pl.multiple_of(cid * _CH, 8)
        chunks.append(off)
    T = len(chunks)

    def copy_in(c):
        b = c % _NBUF
        return pltpu.make_async_copy(
            i_src.at[pl.ds(chunks[c], _CH), :], buf.at[b], sem_in[b])

    def copy_out(c):
        b = c % _NBUF
        return pltpu.make_async_copy(
            buf.at[b], i_dst.at[pl.ds(chunks[c], _CH), :], sem_out[b])

    copy_in(0).start()
    copy_in(1).start()
    for c in range(T):
        if c >= 1 and c + 1 < T:
            copy_out(c - 1).wait()
            copy_in(c + 1).start()
        copy_in(c).wait()
        copy_out(c).start()
    copy_out(T - 2).wait()
    copy_out(T - 1).wait()


_R = 10000
_TNBUF = 8
_TLAG = 4


def _tc_copy_body(u_src, u_dst, buf, sem_in, sem_out):
    T = 100000 // _R

    def copy_in(c):
        b = c % _TNBUF
        return pltpu.make_async_copy(
            u_src.at[pl.ds(c * _R, _R), :], buf.at[b], sem_in.at[b])

    def copy_out(c):
        b = c % _TNBUF
        return pltpu.make_async_copy(
            buf.at[b], u_dst.at[pl.ds(c * _R, _R), :], sem_out.at[b])

    out_waited = [False] * T
    for b in range(min(_TNBUF, T)):
        copy_in(b).start()
    for c in range(T):
        r = c - _TLAG
        if 0 <= r and r + _TNBUF < T:
            copy_out(r).wait()
            out_waited[r] = True
            copy_in(r + _TNBUF).start()
        copy_in(c).wait()
        copy_out(c).start()
    for c in range(T):
        if not out_waited[c]:
            copy_out(c).wait()


def kernel(W_user, W_item):
    mesh = plsc.VectorSubcoreMesh(core_axis_name="c", subcore_axis_name="s")
    item_out = pl.kernel(
        _sc_copy_body,
        out_type=jax.ShapeDtypeStruct(W_item.shape, W_item.dtype),
        mesh=mesh,
        scratch_types=[
            pltpu.VMEM((_NBUF, _CH, 64), jnp.float32),
            pltpu.SemaphoreType.DMA,
            pltpu.SemaphoreType.DMA,
            pltpu.SemaphoreType.DMA,
            pltpu.SemaphoreType.DMA,
        ],
    )(W_item)
    user_out = pl.pallas_call(
        _tc_copy_body,
        in_specs=[pl.BlockSpec(memory_space=pltpu.HBM)],
        out_specs=pl.BlockSpec(memory_space=pltpu.HBM),
        out_shape=jax.ShapeDtypeStruct(W_user.shape, W_user.dtype),
        scratch_shapes=[
            pltpu.VMEM((_TNBUF, _R, 64), jnp.float32),
            pltpu.SemaphoreType.DMA((_TNBUF,)),
            pltpu.SemaphoreType.DMA((_TNBUF,)),
        ],
    )(W_user)
    return (user_out, item_out)


# speedup vs baseline: 1.0541x; 1.0239x over previous
import jax
import jax.numpy as jnp
from jax.experimental import pallas as pl
from jax.experimental.pallas import tpu as pltpu

_R = 2000
_NBUF = 48
_LAG = 24


def _ring_copy_body(u_src, i_src, u_dst, i_dst, buf, sem_in, sem_out):
    chunks = []
    for c in range(100000 // _R):
        chunks.append((u_src, u_dst, c * _R))
    for c in range(1000000 // _R):
        chunks.append((i_src, i_dst, c * _R))
    T = len(chunks)

    def copy_in(c):
        s, _, off = chunks[c]
        b = c % _NBUF
        return pltpu.make_async_copy(s.at[pl.ds(off, _R), :], buf.at[b], sem_in.at[b])

    def copy_out(c):
        _, d, off = chunks[c]
        b = c % _NBUF
        return pltpu.make_async_copy(buf.at[b], d.at[pl.ds(off, _R), :], sem_out.at[b])

    out_waited = [False] * T
    for b in range(min(_NBUF, T)):
        copy_in(b).start()
    for c in range(T):
        r = c - _LAG
        if 0 <= r and r + _NBUF < T:
            copy_out(r).wait()
            out_waited[r] = True
            copy_in(r + _NBUF).start()
        copy_in(c).wait()
        copy_out(c).start()
    for c in range(T):
        if not out_waited[c]:
            copy_out(c).wait()


def kernel(W_user, W_item):
    out = pl.pallas_call(
        _ring_copy_body,
        in_specs=[
            pl.BlockSpec(memory_space=pltpu.HBM),
            pl.BlockSpec(memory_space=pltpu.HBM),
        ],
        out_specs=[
            pl.BlockSpec(memory_space=pltpu.HBM),
            pl.BlockSpec(memory_space=pltpu.HBM),
        ],
        out_shape=[
            jax.ShapeDtypeStruct(W_user.shape, W_user.dtype),
            jax.ShapeDtypeStruct(W_item.shape, W_item.dtype),
        ],
        scratch_shapes=[
            pltpu.VMEM((_NBUF, _R, 64), jnp.float32),
            pltpu.SemaphoreType.DMA((_NBUF,)),
            pltpu.SemaphoreType.DMA((_NBUF,)),
        ],
    )(W_user, W_item)
    return (out[0], out[1])
